# baseline (speedup 1.0000x reference)
import functools

import jax
import jax.numpy as jnp
from jax import lax
from jax.experimental import pallas as pl
from jax.experimental.pallas import tpu as pltpu
from jax.experimental.pallas import tpu_sc as plsc

N = 50000
C = 256
K = 5
H = 128
EPS = 1e-5
SLOPE = 0.01

NC, NS = 2, 16
NW = NC * NS
CH = 32
ROWS_W = 1664
NCHUNK = ROWS_W // CH
NPAD = NW * ROWS_W

T1 = 1000
GRID = N // T1


def _lrelu(x):
    return jnp.where(x >= 0, x, SLOPE * x)


def _tc1_body(f_ref, w_ref, b1_ref, b3_ref,
              ps_ref, q0_ref, q1_ref, q2_ref, q3_ref, q4_ref, r_ref,
              sr_ref, ssr_ref):
    x = f_ref[...]
    y = jnp.dot(x, w_ref[...], preferred_element_type=jnp.float32)
    ps_ref[...] = y[:, 0:H] + b1_ref[...]
    q0_ref[...] = y[:, H:2 * H]
    q1_ref[...] = y[:, 2 * H:3 * H]
    q2_ref[...] = y[:, 3 * H:4 * H]
    q3_ref[...] = y[:, 4 * H:5 * H]
    q4_ref[...] = y[:, 5 * H:6 * H]
    r = y[:, 6 * H:7 * H] + b3_ref[...]
    r_ref[...] = r.astype(jnp.bfloat16)

    i = pl.program_id(0)

    @pl.when(i == 0)
    def _():
        sr_ref[...] = jnp.zeros_like(sr_ref)
        ssr_ref[...] = jnp.zeros_like(ssr_ref)

    sr_ref[...] += jnp.sum(r, axis=0, keepdims=True)
    ssr_ref[...] += jnp.sum(r * r, axis=0, keepdims=True)


def _tc1(f, wbig1, b1, b3):
    row_spec = pl.BlockSpec((T1, H), lambda i: (i, 0))
    return pl.pallas_call(
        _tc1_body,
        grid=(GRID,),
        in_specs=[
            pl.BlockSpec((T1, C), lambda i: (i, 0)),
            pl.BlockSpec((C, 7 * H), lambda i: (0, 0)),
            pl.BlockSpec((1, H), lambda i: (0, 0)),
            pl.BlockSpec((1, H), lambda i: (0, 0)),
        ],
        out_specs=[row_spec] * 7 + [pl.BlockSpec((1, H), lambda i: (0, 0))] * 2,
        out_shape=[jax.ShapeDtypeStruct((NPAD, H), jnp.float32)] * 6
        + [jax.ShapeDtypeStruct((NPAD, H), jnp.bfloat16)]
        + [jax.ShapeDtypeStruct((1, H), jnp.float32)] * 2,
    )(f, wbig1, b1, b3)


def _sc_body(ps_hbm, t0, t1, t2, t3, t4, idx_hbm,
             y_hbm, sum_hbm, ssq_hbm,
             idx_v, bufs, outbuf, sum_v, ssq_v, sem0, sem1, wsem0, wsem1):
    wid = lax.axis_index("s") * NC + lax.axis_index("c")
    base = wid * ROWS_W
    tables = (t0, t1, t2, t3, t4)
    sems = (sem0, sem1)
    wsems = (wsem0, wsem1)
    NJ = H // 16

    pltpu.sync_copy(idx_hbm.at[wid], idx_v)

    def copies(c, s):
        row0 = base + c * CH
        ops = [pltpu.make_async_copy(tables[k].at[idx_v.at[c, k]],
                                     bufs.at[s, k], sems[s]) for k in range(K)]
        ops.append(pltpu.make_async_copy(ps_hbm.at[pl.ds(row0, CH)],
                                         bufs.at[s, K], sems[s]))
        return ops

    def issue(c, s):
        for op in copies(c, s):
            op.start()

    def wcopy(c, s):
        row0 = base + c * CH
        return pltpu.make_async_copy(outbuf.at[s],
                                     y_hbm.at[pl.ds(row0, CH)], wsems[s])

    def compute(c, s, stats):
        row0 = base + c * CH

        @pl.when(c >= 2)
        def _():
            wcopy(c - 2, s).wait()

        for op in copies(c, s):
            op.wait()

        def row_body(r, st):
            valid = (row0 + r) < N
            sums, ssqs = st
            new_sums, new_ssqs = [], []
            for j in range(NJ):
                sl = pl.ds(16 * j, 16)
                s_ = bufs[s, K, r, sl]
                for k in range(K):
                    s_ = s_ + bufs[s, k, r, sl]
                outbuf[s, r, sl] = s_
                sv = jnp.where(valid, s_, 0.0)
                new_sums.append(sums[j] + sv)
                new_ssqs.append(ssqs[j] + sv * sv)
            return (tuple(new_sums), tuple(new_ssqs))

        stats = plsc.parallel_loop(0, CH, 1, unroll=4, carry=stats)(row_body)
        wcopy(c, s).start()
        return stats

    zeros = tuple(jnp.zeros((16,), jnp.float32) for _ in range(NJ))
    stats0 = (zeros, zeros)

    issue(0, 0)
    issue(1, 1)

    def pair_body(i, stats):
        c = 2 * i
        stats = compute(c, 0, stats)

        @pl.when(c + 2 < NCHUNK)
        def _():
            issue(c + 2, 0)

        stats = compute(c + 1, 1, stats)

        @pl.when(c + 3 < NCHUNK)
        def _():
            issue(c + 3, 1)

        return stats

    stats = lax.fori_loop(0, NCHUNK // 2, pair_body, stats0)
    wcopy(NCHUNK - 2, 0).wait()
    wcopy(NCHUNK - 1, 1).wait()
    sums, ssqs = stats
    for j in range(NJ):
        sl = pl.ds(16 * j, 16)
        sum_v[sl] = sums[j]
        ssq_v[sl] = ssqs[j]
    pltpu.sync_copy(sum_v, sum_hbm.at[wid])
    pltpu.sync_copy(ssq_v, ssq_hbm.at[wid])


_sc_gather_sum = functools.partial(
    pl.kernel,
    out_type=[
        jax.ShapeDtypeStruct((NPAD, H), jnp.float32),
        jax.ShapeDtypeStruct((NW, H), jnp.float32),
        jax.ShapeDtypeStruct((NW, H), jnp.float32),
    ],
    mesh=plsc.VectorSubcoreMesh(core_axis_name="c", subcore_axis_name="s"),
    scratch_types=[
        pltpu.VMEM((NCHUNK, K, CH), jnp.int32),
        pltpu.VMEM((2, K + 1, CH, H), jnp.float32),
        pltpu.VMEM((2, CH, H), jnp.float32),
        pltpu.VMEM((H,), jnp.float32),
        pltpu.VMEM((H,), jnp.float32),
        pltpu.SemaphoreType.DMA,
        pltpu.SemaphoreType.DMA,
        pltpu.SemaphoreType.DMA,
        pltpu.SemaphoreType.DMA,
    ],
)(_sc_body)


def _tc2_body(y1_ref, s_ref, ss_ref, w_ref, b2_ref,
              ps_ref, q0_ref, q1_ref, q2_ref, q3_ref, q4_ref):
    s = jnp.sum(s_ref[...], axis=0, keepdims=True)
    ss = jnp.sum(ss_ref[...], axis=0, keepdims=True)
    mean = s * (1.0 / N)
    var = ss * (1.0 / N) - mean * mean
    inv = lax.rsqrt(var + EPS)
    x1 = _lrelu((y1_ref[...] - mean) * inv)
    y = jnp.dot(x1, w_ref[...], preferred_element_type=jnp.float32)
    ps_ref[...] = y[:, 0:H] + b2_ref[...]
    q0_ref[...] = y[:, H:2 * H]
    q1_ref[...] = y[:, 2 * H:3 * H]
    q2_ref[...] = y[:, 3 * H:4 * H]
    q3_ref[...] = y[:, 4 * H:5 * H]
    q4_ref[...] = y[:, 5 * H:6 * H]


def _tc2(y1, s1, ss1, wbig2, b2):
    row_spec = pl.BlockSpec((T1, H), lambda i: (i, 0))
    return pl.pallas_call(
        _tc2_body,
        grid=(GRID,),
        in_specs=[
            pl.BlockSpec((T1, H), lambda i: (i, 0)),
            pl.BlockSpec((NW, H), lambda i: (0, 0)),
            pl.BlockSpec((NW, H), lambda i: (0, 0)),
            pl.BlockSpec((H, 6 * H), lambda i: (0, 0)),
            pl.BlockSpec((1, H), lambda i: (0, 0)),
        ],
        out_specs=[row_spec] * 6,
        out_shape=[jax.ShapeDtypeStruct((NPAD, H), jnp.float32)] * 6,
    )(y1, s1, ss1, wbig2, b2)


def _tc3_body(y2_ref, s2_ref, ss2_ref, r_ref, sr_ref, ssr_ref, w4_ref, o_ref):
    s2 = jnp.sum(s2_ref[...], axis=0, keepdims=True)
    ss2 = jnp.sum(ss2_ref[...], axis=0, keepdims=True)
    m2 = s2 * (1.0 / N)
    v2 = ss2 * (1.0 / N) - m2 * m2
    inv2 = lax.rsqrt(v2 + EPS)
    mr = sr_ref[...] * (1.0 / N)
    vr = ssr_ref[...] * (1.0 / N) - mr * mr
    invr = lax.rsqrt(vr + EPS)
    x = _lrelu((y2_ref[...] - m2) * inv2
               + (r_ref[...].astype(jnp.float32) - mr) * invr)
    o_ref[...] = jnp.sum(x * w4_ref[...], axis=1, keepdims=True)


def _tc3(y2, s2, ss2, r, sr, ssr, w4row):
    return pl.pallas_call(
        _tc3_body,
        grid=(GRID,),
        in_specs=[
            pl.BlockSpec((T1, H), lambda i: (i, 0)),
            pl.BlockSpec((NW, H), lambda i: (0, 0)),
            pl.BlockSpec((NW, H), lambda i: (0, 0)),
            pl.BlockSpec((T1, H), lambda i: (i, 0)),
            pl.BlockSpec((1, H), lambda i: (0, 0)),
            pl.BlockSpec((1, H), lambda i: (0, 0)),
            pl.BlockSpec((1, H), lambda i: (0, 0)),
        ],
        out_specs=pl.BlockSpec((T1, 1), lambda i: (i, 0)),
        out_shape=jax.ShapeDtypeStruct((N, 1), jnp.float32),
    )(y2, s2, ss2, r, sr, ssr, w4row)


def kernel(features, neighbors_index, W1, b1, W2, b2, W3, b3, W4):
    f = features[0]

    idx = neighbors_index[0].astype(jnp.int32).T
    idx = jnp.pad(idx, ((0, 0), (0, NPAD - N)))
    idxr = idx.reshape(K, NW, NCHUNK, CH).transpose(1, 2, 0, 3)

    wb1 = [W1[:, j * C:(j + 1) * C].T for j in range(K + 1)] + [W3.T]
    wbig1 = jnp.concatenate(wb1, axis=1)
    wb2 = [W2[:, j * H:(j + 1) * H].T for j in range(K + 1)]
    wbig2 = jnp.concatenate(wb2, axis=1)

    b1r = b1.reshape(1, H)
    b2r = b2.reshape(1, H)
    b3r = b3.reshape(1, H)
    w4r = W4.reshape(1, H)

    ps1, q0, q1, q2, q3, q4, r, sr, ssr = _tc1(f, wbig1, b1r, b3r)
    y1, s1, ss1 = _sc_gather_sum(ps1, q0, q1, q2, q3, q4, idxr)
    ps2, p0, p1, p2, p3, p4 = _tc2(y1, s1, ss1, wbig2, b2r)
    y2, s2, ss2 = _sc_gather_sum(ps2, p0, p1, p2, p3, p4, idxr)
    out = _tc3(y2, s2, ss2, r, sr, ssr, w4r)
    return out.reshape(1, N, 1)


# speedup vs baseline: 1.0701x; 1.0701x over previous
import functools

import jax
import jax.numpy as jnp
from jax import lax
from jax.experimental import pallas as pl
from jax.experimental.pallas import tpu as pltpu
from jax.experimental.pallas import tpu_sc as plsc

N = 50000
C = 256
K = 5
H = 128
EPS = 1e-5
SLOPE = 0.01

NC, NS = 2, 16
NW = NC * NS
CH = 64
ROWS_W = 1664
NCHUNK = ROWS_W // CH
NPAD = NW * ROWS_W

T1 = 1000
GRID = N // T1


def _lrelu(x):
    return jnp.where(x >= 0, x, SLOPE * x)


def _tc1_body(f_ref, w_ref, b1_ref, b3_ref,
              ps_ref, q0_ref, q1_ref, q2_ref, q3_ref, q4_ref, r_ref,
              sr_ref, ssr_ref):
    x = f_ref[...]
    y = jnp.dot(x, w_ref[...], preferred_element_type=jnp.float32)
    ps_ref[...] = y[:, 0:H] + b1_ref[...]
    q0_ref[...] = y[:, H:2 * H]
    q1_ref[...] = y[:, 2 * H:3 * H]
    q2_ref[...] = y[:, 3 * H:4 * H]
    q3_ref[...] = y[:, 4 * H:5 * H]
    q4_ref[...] = y[:, 5 * H:6 * H]
    r = y[:, 6 * H:7 * H] + b3_ref[...]
    r_ref[...] = r.astype(jnp.bfloat16)

    i = pl.program_id(0)

    @pl.when(i == 0)
    def _():
        sr_ref[...] = jnp.zeros_like(sr_ref)
        ssr_ref[...] = jnp.zeros_like(ssr_ref)

    sr_ref[...] += jnp.sum(r, axis=0, keepdims=True)
    ssr_ref[...] += jnp.sum(r * r, axis=0, keepdims=True)


def _tc1(f, wbig1, b1, b3):
    row_spec = pl.BlockSpec((T1, H), lambda i: (i, 0))
    return pl.pallas_call(
        _tc1_body,
        grid=(GRID,),
        in_specs=[
            pl.BlockSpec((T1, C), lambda i: (i, 0)),
            pl.BlockSpec((C, 7 * H), lambda i: (0, 0)),
            pl.BlockSpec((1, H), lambda i: (0, 0)),
            pl.BlockSpec((1, H), lambda i: (0, 0)),
        ],
        out_specs=[row_spec] * 7 + [pl.BlockSpec((1, H), lambda i: (0, 0))] * 2,
        out_shape=[jax.ShapeDtypeStruct((NPAD, H), jnp.float32)] * 6
        + [jax.ShapeDtypeStruct((NPAD, H), jnp.bfloat16)]
        + [jax.ShapeDtypeStruct((1, H), jnp.float32)] * 2,
    )(f, wbig1, b1, b3)


def _sc_body(ps_hbm, t0, t1, t2, t3, t4, idx_hbm,
             y_hbm, sum_hbm, ssq_hbm,
             idx_v, bufs, sum_v, ssq_v, sem0, sem1):
    wid = lax.axis_index("s") * NC + lax.axis_index("c")
    base = wid * ROWS_W
    tables = (t0, t1, t2, t3, t4)
    sems = (sem0, sem1)
    NJ = H // 16

    pltpu.sync_copy(idx_hbm.at[wid], idx_v)

    def copies(c, s):
        row0 = base + c * CH
        ops = [pltpu.make_async_copy(tables[k].at[idx_v.at[c, k]],
                                     bufs.at[s, k], sems[s]) for k in range(K)]
        ops.append(pltpu.make_async_copy(ps_hbm.at[pl.ds(row0, CH)],
                                         bufs.at[s, K], sems[s]))
        return ops

    def issue(c, s):
        for op in copies(c, s):
            op.start()

    def compute(c, s, stats):
        row0 = base + c * CH
        for op in copies(c, s):
            op.wait()

        def row_body(r, st):
            valid = (row0 + r) < N
            sums, ssqs = st
            new_sums, new_ssqs = [], []
            for j in range(NJ):
                sl = pl.ds(16 * j, 16)
                s_ = bufs[s, K, r, sl]
                for k in range(K):
                    s_ = s_ + bufs[s, k, r, sl]
                bufs[s, K, r, sl] = s_
                sv = jnp.where(valid, s_, 0.0)
                new_sums.append(sums[j] + sv)
                new_ssqs.append(ssqs[j] + sv * sv)
            return (tuple(new_sums), tuple(new_ssqs))

        stats = plsc.parallel_loop(0, CH, 1, unroll=4, carry=stats)(row_body)
        pltpu.sync_copy(bufs.at[s, K], y_hbm.at[pl.ds(row0, CH)])
        return stats

    zeros = tuple(jnp.zeros((16,), jnp.float32) for _ in range(NJ))
    stats0 = (zeros, zeros)

    issue(0, 0)
    issue(1, 1)

    def pair_body(i, stats):
        c = 2 * i
        stats = compute(c, 0, stats)

        @pl.when(c + 2 < NCHUNK)
        def _():
            issue(c + 2, 0)

        stats = compute(c + 1, 1, stats)

        @pl.when(c + 3 < NCHUNK)
        def _():
            issue(c + 3, 1)

        return stats

    stats = lax.fori_loop(0, NCHUNK // 2, pair_body, stats0)
    sums, ssqs = stats
    for j in range(NJ):
        sl = pl.ds(16 * j, 16)
        sum_v[sl] = sums[j]
        ssq_v[sl] = ssqs[j]
    pltpu.sync_copy(sum_v, sum_hbm.at[wid])
    pltpu.sync_copy(ssq_v, ssq_hbm.at[wid])


_sc_gather_sum = functools.partial(
    pl.kernel,
    out_type=[
        jax.ShapeDtypeStruct((NPAD, H), jnp.float32),
        jax.ShapeDtypeStruct((NW, H), jnp.float32),
        jax.ShapeDtypeStruct((NW, H), jnp.float32),
    ],
    mesh=plsc.VectorSubcoreMesh(core_axis_name="c", subcore_axis_name="s"),
    scratch_types=[
        pltpu.VMEM((NCHUNK, K, CH), jnp.int32),
        pltpu.VMEM((2, K + 1, CH, H), jnp.float32),
        pltpu.VMEM((H,), jnp.float32),
        pltpu.VMEM((H,), jnp.float32),
        pltpu.SemaphoreType.DMA,
        pltpu.SemaphoreType.DMA,
    ],
)(_sc_body)


def _tc2_body(y1_ref, s_ref, ss_ref, w_ref, b2_ref,
              ps_ref, q0_ref, q1_ref, q2_ref, q3_ref, q4_ref):
    s = jnp.sum(s_ref[...], axis=0, keepdims=True)
    ss = jnp.sum(ss_ref[...], axis=0, keepdims=True)
    mean = s * (1.0 / N)
    var = ss * (1.0 / N) - mean * mean
    inv = lax.rsqrt(var + EPS)
    x1 = _lrelu((y1_ref[...] - mean) * inv)
    y = jnp.dot(x1, w_ref[...], preferred_element_type=jnp.float32)
    ps_ref[...] = y[:, 0:H] + b2_ref[...]
    q0_ref[...] = y[:, H:2 * H]
    q1_ref[...] = y[:, 2 * H:3 * H]
    q2_ref[...] = y[:, 3 * H:4 * H]
    q3_ref[...] = y[:, 4 * H:5 * H]
    q4_ref[...] = y[:, 5 * H:6 * H]


def _tc2(y1, s1, ss1, wbig2, b2):
    row_spec = pl.BlockSpec((T1, H), lambda i: (i, 0))
    return pl.pallas_call(
        _tc2_body,
        grid=(GRID,),
        in_specs=[
            pl.BlockSpec((T1, H), lambda i: (i, 0)),
            pl.BlockSpec((NW, H), lambda i: (0, 0)),
            pl.BlockSpec((NW, H), lambda i: (0, 0)),
            pl.BlockSpec((H, 6 * H), lambda i: (0, 0)),
            pl.BlockSpec((1, H), lambda i: (0, 0)),
        ],
        out_specs=[row_spec] * 6,
        out_shape=[jax.ShapeDtypeStruct((NPAD, H), jnp.float32)] * 6,
    )(y1, s1, ss1, wbig2, b2)


def _tc3_body(y2_ref, s2_ref, ss2_ref, r_ref, sr_ref, ssr_ref, w4_ref, o_ref):
    s2 = jnp.sum(s2_ref[...], axis=0, keepdims=True)
    ss2 = jnp.sum(ss2_ref[...], axis=0, keepdims=True)
    m2 = s2 * (1.0 / N)
    v2 = ss2 * (1.0 / N) - m2 * m2
    inv2 = lax.rsqrt(v2 + EPS)
    mr = sr_ref[...] * (1.0 / N)
    vr = ssr_ref[...] * (1.0 / N) - mr * mr
    invr = lax.rsqrt(vr + EPS)
    x = _lrelu((y2_ref[...] - m2) * inv2
               + (r_ref[...].astype(jnp.float32) - mr) * invr)
    o_ref[...] = jnp.sum(x * w4_ref[...], axis=1, keepdims=True)


def _tc3(y2, s2, ss2, r, sr, ssr, w4row):
    return pl.pallas_call(
        _tc3_body,
        grid=(GRID,),
        in_specs=[
            pl.BlockSpec((T1, H), lambda i: (i, 0)),
            pl.BlockSpec((NW, H), lambda i: (0, 0)),
            pl.BlockSpec((NW, H), lambda i: (0, 0)),
            pl.BlockSpec((T1, H), lambda i: (i, 0)),
            pl.BlockSpec((1, H), lambda i: (0, 0)),
            pl.BlockSpec((1, H), lambda i: (0, 0)),
            pl.BlockSpec((1, H), lambda i: (0, 0)),
        ],
        out_specs=pl.BlockSpec((T1, 1), lambda i: (i, 0)),
        out_shape=jax.ShapeDtypeStruct((N, 1), jnp.float32),
    )(y2, s2, ss2, r, sr, ssr, w4row)


def kernel(features, neighbors_index, W1, b1, W2, b2, W3, b3, W4):
    f = features[0]

    idx = neighbors_index[0].astype(jnp.int32).T
    idx = jnp.pad(idx, ((0, 0), (0, NPAD - N)))
    idxr = idx.reshape(K, NW, NCHUNK, CH).transpose(1, 2, 0, 3)

    wb1 = [W1[:, j * C:(j + 1) * C].T for j in range(K + 1)] + [W3.T]
    wbig1 = jnp.concatenate(wb1, axis=1)
    wb2 = [W2[:, j * H:(j + 1) * H].T for j in range(K + 1)]
    wbig2 = jnp.concatenate(wb2, axis=1)

    b1r = b1.reshape(1, H)
    b2r = b2.reshape(1, H)
    b3r = b3.reshape(1, H)
    w4r = W4.reshape(1, H)

    ps1, q0, q1, q2, q3, q4, r, sr, ssr = _tc1(f, wbig1, b1r, b3r)
    y1, s1, ss1 = _sc_gather_sum(ps1, q0, q1, q2, q3, q4, idxr)
    ps2, p0, p1, p2, p3, p4 = _tc2(y1, s1, ss1, wbig2, b2r)
    y2, s2, ss2 = _sc_gather_sum(ps2, p0, p1, p2, p3, p4, idxr)
    out = _tc3(y2, s2, ss2, r, sr, ssr, w4r)
    return out.reshape(1, N, 1)


# speedup vs baseline: 1.0803x; 1.0096x over previous
import functools

import jax
import jax.numpy as jnp
from jax import lax
from jax.experimental import pallas as pl
from jax.experimental.pallas import tpu as pltpu
from jax.experimental.pallas import tpu_sc as plsc

N = 50000
C = 256
K = 5
H = 128
EPS = 1e-5
SLOPE = 0.01

NC, NS = 2, 16
NW = NC * NS
CH = 64
ROWS_W = 1664
NCHUNK = ROWS_W // CH
NPAD = NW * ROWS_W

T1 = 1000
GRID = N // T1


def _lrelu(x):
    return jnp.where(x >= 0, x, SLOPE * x)


def _tc1_body(f_ref, w_ref, b1_ref, b3_ref,
              ps_ref, q0_ref, q1_ref, q2_ref, q3_ref, q4_ref, r_ref,
              sr_ref, ssr_ref):
    x = f_ref[...]
    y = jnp.dot(x, w_ref[...], preferred_element_type=jnp.float32)
    ps_ref[...] = y[:, 0:H] + b1_ref[...]
    q0_ref[...] = y[:, H:2 * H]
    q1_ref[...] = y[:, 2 * H:3 * H]
    q2_ref[...] = y[:, 3 * H:4 * H]
    q3_ref[...] = y[:, 4 * H:5 * H]
    q4_ref[...] = y[:, 5 * H:6 * H]
    r = y[:, 6 * H:7 * H] + b3_ref[...]
    r_ref[...] = r.astype(jnp.bfloat16)

    i = pl.program_id(0)

    @pl.when(i == 0)
    def _():
        sr_ref[...] = jnp.zeros_like(sr_ref)
        ssr_ref[...] = jnp.zeros_like(ssr_ref)

    sr_ref[...] += jnp.sum(r, axis=0, keepdims=True)
    ssr_ref[...] += jnp.sum(r * r, axis=0, keepdims=True)


def _tc1(f, wbig1, b1, b3):
    row_spec = pl.BlockSpec((T1, H), lambda i: (i, 0))
    return pl.pallas_call(
        _tc1_body,
        grid=(GRID,),
        in_specs=[
            pl.BlockSpec((T1, C), lambda i: (i, 0)),
            pl.BlockSpec((C, 7 * H), lambda i: (0, 0)),
            pl.BlockSpec((1, H), lambda i: (0, 0)),
            pl.BlockSpec((1, H), lambda i: (0, 0)),
        ],
        out_specs=[row_spec] * 7 + [pl.BlockSpec((1, H), lambda i: (0, 0))] * 2,
        out_shape=[jax.ShapeDtypeStruct((NPAD, H), jnp.float32)] * 6
        + [jax.ShapeDtypeStruct((NPAD, H), jnp.bfloat16)]
        + [jax.ShapeDtypeStruct((1, H), jnp.float32)] * 2,
    )(f, wbig1, b1, b3)


def _sc_body(ps_hbm, t0, t1, t2, t3, t4, idx_hbm,
             y_hbm, sum_hbm, ssq_hbm,
             idx_v, bufs, sum_v, ssq_v, sem0, sem1):
    wid = lax.axis_index("s") * NC + lax.axis_index("c")
    base = wid * ROWS_W
    tables = (t0, t1, t2, t3, t4)
    sems = (sem0, sem1)
    NJ = H // 16

    pltpu.sync_copy(idx_hbm.at[wid], idx_v)

    def copies(c, s):
        row0 = base + c * CH
        ops = [pltpu.make_async_copy(tables[k].at[idx_v.at[c, k]],
                                     bufs.at[s, k], sems[s]) for k in range(K)]
        ops.append(pltpu.make_async_copy(ps_hbm.at[pl.ds(row0, CH)],
                                         bufs.at[s, K], sems[s]))
        return ops

    def issue(c, s):
        for op in copies(c, s):
            op.start()

    def compute(c, s, stats):
        row0 = base + c * CH
        for op in copies(c, s):
            op.wait()

        def row_body(r, st):
            valid = (row0 + r) < N
            sums, ssqs = st
            new_sums, new_ssqs = [], []
            for j in range(NJ):
                sl = pl.ds(16 * j, 16)
                s_ = bufs[s, K, r, sl]
                for k in range(K):
                    s_ = s_ + bufs[s, k, r, sl]
                bufs[s, K, r, sl] = s_
                sv = jnp.where(valid, s_, 0.0)
                new_sums.append(sums[j] + sv)
                new_ssqs.append(ssqs[j] + sv * sv)
            return (tuple(new_sums), tuple(new_ssqs))

        stats = plsc.parallel_loop(0, CH, 1, unroll=4, carry=stats)(row_body)
        pltpu.sync_copy(bufs.at[s, K], y_hbm.at[pl.ds(row0, CH)])
        return stats

    zeros = tuple(jnp.zeros((16,), jnp.float32) for _ in range(NJ))
    stats0 = (zeros, zeros)

    issue(0, 0)
    issue(1, 1)

    def pair_body(i, stats):
        c = 2 * i
        stats = compute(c, 0, stats)

        @pl.when(c + 2 < NCHUNK)
        def _():
            issue(c + 2, 0)

        stats = compute(c + 1, 1, stats)

        @pl.when(c + 3 < NCHUNK)
        def _():
            issue(c + 3, 1)

        return stats

    stats = lax.fori_loop(0, NCHUNK // 2, pair_body, stats0)
    sums, ssqs = stats
    for j in range(NJ):
        sl = pl.ds(16 * j, 16)
        sum_v[sl] = sums[j]
        ssq_v[sl] = ssqs[j]
    pltpu.sync_copy(sum_v, sum_hbm.at[wid])
    pltpu.sync_copy(ssq_v, ssq_hbm.at[wid])


_sc_gather_sum = functools.partial(
    pl.kernel,
    out_type=[
        jax.ShapeDtypeStruct((NPAD, H), jnp.float32),
        jax.ShapeDtypeStruct((NW, H), jnp.float32),
        jax.ShapeDtypeStruct((NW, H), jnp.float32),
    ],
    mesh=plsc.VectorSubcoreMesh(core_axis_name="c", subcore_axis_name="s"),
    scratch_types=[
        pltpu.VMEM((NCHUNK, K, CH), jnp.int32),
        pltpu.VMEM((2, K + 1, CH, H), jnp.float32),
        pltpu.VMEM((H,), jnp.float32),
        pltpu.VMEM((H,), jnp.float32),
        pltpu.SemaphoreType.DMA,
        pltpu.SemaphoreType.DMA,
    ],
)(_sc_body)


def _tc2_body(y1_ref, s_ref, ss_ref, w_ref, b2_ref,
              ps_ref, q0_ref, q1_ref, q2_ref, q3_ref, q4_ref):
    s = jnp.sum(s_ref[...], axis=0, keepdims=True)
    ss = jnp.sum(ss_ref[...], axis=0, keepdims=True)
    mean = s * (1.0 / N)
    var = ss * (1.0 / N) - mean * mean
    inv = lax.rsqrt(var + EPS)
    x1 = _lrelu((y1_ref[...] - mean) * inv)
    y = jnp.dot(x1, w_ref[...], preferred_element_type=jnp.float32)
    ps_ref[...] = y[:, 0:H] + b2_ref[...]
    q0_ref[...] = y[:, H:2 * H]
    q1_ref[...] = y[:, 2 * H:3 * H]
    q2_ref[...] = y[:, 3 * H:4 * H]
    q3_ref[...] = y[:, 4 * H:5 * H]
    q4_ref[...] = y[:, 5 * H:6 * H]


def _tc2(y1, s1, ss1, wbig2, b2):
    row_spec = pl.BlockSpec((T1, H), lambda i: (i, 0))
    return pl.pallas_call(
        _tc2_body,
        grid=(GRID,),
        in_specs=[
            pl.BlockSpec((T1, H), lambda i: (i, 0)),
            pl.BlockSpec((NW, H), lambda i: (0, 0)),
            pl.BlockSpec((NW, H), lambda i: (0, 0)),
            pl.BlockSpec((H, 6 * H), lambda i: (0, 0)),
            pl.BlockSpec((1, H), lambda i: (0, 0)),
        ],
        out_specs=[row_spec] * 6,
        out_shape=[jax.ShapeDtypeStruct((NPAD, H), jnp.float32)] * 6,
    )(y1, s1, ss1, wbig2, b2)


def _tc3_body(y2_ref, s2_ref, ss2_ref, r_ref, sr_ref, ssr_ref, w4_ref, o_ref):
    s2 = jnp.sum(s2_ref[...], axis=0, keepdims=True)
    ss2 = jnp.sum(ss2_ref[...], axis=0, keepdims=True)
    m2 = s2 * (1.0 / N)
    v2 = ss2 * (1.0 / N) - m2 * m2
    inv2 = lax.rsqrt(v2 + EPS)
    mr = sr_ref[...] * (1.0 / N)
    vr = ssr_ref[...] * (1.0 / N) - mr * mr
    invr = lax.rsqrt(vr + EPS)
    x = _lrelu((y2_ref[...] - m2) * inv2
               + (r_ref[...].astype(jnp.float32) - mr) * invr)
    o_ref[...] = jnp.sum(x * w4_ref[...], axis=1, keepdims=True).reshape(1, 1, T1)


def _tc3(y2, s2, ss2, r, sr, ssr, w4row):
    return pl.pallas_call(
        _tc3_body,
        grid=(GRID,),
        in_specs=[
            pl.BlockSpec((T1, H), lambda i: (i, 0)),
            pl.BlockSpec((NW, H), lambda i: (0, 0)),
            pl.BlockSpec((NW, H), lambda i: (0, 0)),
            pl.BlockSpec((T1, H), lambda i: (i, 0)),
            pl.BlockSpec((1, H), lambda i: (0, 0)),
            pl.BlockSpec((1, H), lambda i: (0, 0)),
            pl.BlockSpec((1, H), lambda i: (0, 0)),
        ],
        out_specs=pl.BlockSpec((1, 1, T1), lambda i: (i, 0, 0)),
        out_shape=jax.ShapeDtypeStruct((GRID, 1, T1), jnp.float32),
    )(y2, s2, ss2, r, sr, ssr, w4row)


def kernel(features, neighbors_index, W1, b1, W2, b2, W3, b3, W4):
    f = features[0]

    idx = neighbors_index[0].astype(jnp.int32).T
    idx = jnp.pad(idx, ((0, 0), (0, NPAD - N)))
    idxr = idx.reshape(K, NW, NCHUNK, CH).transpose(1, 2, 0, 3)

    wb1 = [W1[:, j * C:(j + 1) * C].T for j in range(K + 1)] + [W3.T]
    wbig1 = jnp.concatenate(wb1, axis=1)
    wb2 = [W2[:, j * H:(j + 1) * H].T for j in range(K + 1)]
    wbig2 = jnp.concatenate(wb2, axis=1)

    b1r = b1.reshape(1, H)
    b2r = b2.reshape(1, H)
    b3r = b3.reshape(1, H)
    w4r = W4.reshape(1, H)

    ps1, q0, q1, q2, q3, q4, r, sr, ssr = _tc1(f, wbig1, b1r, b3r)
    y1, s1, ss1 = _sc_gather_sum(ps1, q0, q1, q2, q3, q4, idxr)
    ps2, p0, p1, p2, p3, p4 = _tc2(y1, s1, ss1, wbig2, b2r)
    y2, s2, ss2 = _sc_gather_sum(ps2, p0, p1, p2, p3, p4, idxr)
    out = _tc3(y2, s2, ss2, r, sr, ssr, w4r)
    return out.reshape(1, N, 1)


# speedup vs baseline: 1.0827x; 1.0022x over previous
import functools

import jax
import jax.numpy as jnp
from jax import lax
from jax.experimental import pallas as pl
from jax.experimental.pallas import tpu as pltpu
from jax.experimental.pallas import tpu_sc as plsc

N = 50000
C = 256
K = 5
H = 128
EPS = 1e-5
SLOPE = 0.01

NC, NS = 2, 16
NW = NC * NS
CH = 64
ROWS_W = 1664
NCHUNK = ROWS_W // CH
NPAD = NW * ROWS_W

T1 = 1000
GRID = N // T1


def _lrelu(x):
    return jnp.where(x >= 0, x, SLOPE * x)


def _tc1_body(f_ref, w_ref, b1_ref, b3_ref,
              ps_ref, q0_ref, q1_ref, q2_ref, q3_ref, q4_ref, r_ref,
              sr_ref, ssr_ref):
    x = f_ref[...]
    y = jnp.dot(x, w_ref[...], preferred_element_type=jnp.float32)
    ps_ref[...] = y[:, 0:H] + b1_ref[...]
    q0_ref[...] = y[:, H:2 * H]
    q1_ref[...] = y[:, 2 * H:3 * H]
    q2_ref[...] = y[:, 3 * H:4 * H]
    q3_ref[...] = y[:, 4 * H:5 * H]
    q4_ref[...] = y[:, 5 * H:6 * H]
    r = y[:, 6 * H:7 * H] + b3_ref[...]
    r_ref[...] = r.astype(jnp.bfloat16)

    i = pl.program_id(0)

    @pl.when(i == 0)
    def _():
        sr_ref[...] = jnp.zeros_like(sr_ref)
        ssr_ref[...] = jnp.zeros_like(ssr_ref)

    sr_ref[...] += jnp.sum(r, axis=0, keepdims=True)
    ssr_ref[...] += jnp.sum(r * r, axis=0, keepdims=True)


def _tc1(f, wbig1, b1, b3):
    row_spec = pl.BlockSpec((T1, H), lambda i: (i, 0))
    return pl.pallas_call(
        _tc1_body,
        grid=(GRID,),
        in_specs=[
            pl.BlockSpec((T1, C), lambda i: (i, 0)),
            pl.BlockSpec((C, 7 * H), lambda i: (0, 0)),
            pl.BlockSpec((1, H), lambda i: (0, 0)),
            pl.BlockSpec((1, H), lambda i: (0, 0)),
        ],
        out_specs=[row_spec] * 7 + [pl.BlockSpec((1, H), lambda i: (0, 0))] * 2,
        out_shape=[jax.ShapeDtypeStruct((NPAD, H), jnp.float32)] * 6
        + [jax.ShapeDtypeStruct((NPAD, H), jnp.bfloat16)]
        + [jax.ShapeDtypeStruct((1, H), jnp.float32)] * 2,
    )(f, wbig1, b1, b3)


def _sc_body(ps_hbm, t0, t1, t2, t3, t4, idx_hbm,
             y_hbm, sum_hbm, ssq_hbm,
             idx_v, bufs, sum_v, ssq_v, sem0, sem1):
    wid = lax.axis_index("s") * NC + lax.axis_index("c")
    base = wid * ROWS_W
    tables = (t0, t1, t2, t3, t4)
    sems = (sem0, sem1)
    NJ = H // 16

    pltpu.sync_copy(idx_hbm.at[wid], idx_v)

    def copies(c, s):
        row0 = base + c * CH
        ops = [pltpu.make_async_copy(tables[k].at[idx_v.at[c, k]],
                                     bufs.at[s, k], sems[s]) for k in range(K)]
        ops.append(pltpu.make_async_copy(ps_hbm.at[pl.ds(row0, CH)],
                                         bufs.at[s, K], sems[s]))
        return ops

    def issue(c, s):
        for op in copies(c, s):
            op.start()

    def compute(c, s, stats):
        row0 = base + c * CH
        for op in copies(c, s):
            op.wait()

        def row_body(r, st):
            valid = (row0 + r) < N
            sums, ssqs = st
            new_sums, new_ssqs = [], []
            for j in range(NJ):
                sl = pl.ds(16 * j, 16)
                s_ = bufs[s, K, r, sl]
                for k in range(K):
                    s_ = s_ + bufs[s, k, r, sl]
                bufs[s, K, r, sl] = s_
                sv = jnp.where(valid, s_, 0.0)
                new_sums.append(sums[j] + sv)
                new_ssqs.append(ssqs[j] + sv * sv)
            return (tuple(new_sums), tuple(new_ssqs))

        stats = plsc.parallel_loop(0, CH, 1, unroll=8, carry=stats)(row_body)
        pltpu.sync_copy(bufs.at[s, K], y_hbm.at[pl.ds(row0, CH)])
        return stats

    zeros = tuple(jnp.zeros((16,), jnp.float32) for _ in range(NJ))
    stats0 = (zeros, zeros)

    issue(0, 0)
    issue(1, 1)

    def pair_body(i, stats):
        c = 2 * i
        stats = compute(c, 0, stats)

        @pl.when(c + 2 < NCHUNK)
        def _():
            issue(c + 2, 0)

        stats = compute(c + 1, 1, stats)

        @pl.when(c + 3 < NCHUNK)
        def _():
            issue(c + 3, 1)

        return stats

    stats = lax.fori_loop(0, NCHUNK // 2, pair_body, stats0)
    sums, ssqs = stats
    for j in range(NJ):
        sl = pl.ds(16 * j, 16)
        sum_v[sl] = sums[j]
        ssq_v[sl] = ssqs[j]
    pltpu.sync_copy(sum_v, sum_hbm.at[wid])
    pltpu.sync_copy(ssq_v, ssq_hbm.at[wid])


_sc_gather_sum = functools.partial(
    pl.kernel,
    out_type=[
        jax.ShapeDtypeStruct((NPAD, H), jnp.float32),
        jax.ShapeDtypeStruct((NW, H), jnp.float32),
        jax.ShapeDtypeStruct((NW, H), jnp.float32),
    ],
    mesh=plsc.VectorSubcoreMesh(core_axis_name="c", subcore_axis_name="s"),
    scratch_types=[
        pltpu.VMEM((NCHUNK, K, CH), jnp.int32),
        pltpu.VMEM((2, K + 1, CH, H), jnp.float32),
        pltpu.VMEM((H,), jnp.float32),
        pltpu.VMEM((H,), jnp.float32),
        pltpu.SemaphoreType.DMA,
        pltpu.SemaphoreType.DMA,
    ],
)(_sc_body)


def _tc2_body(y1_ref, s_ref, ss_ref, w_ref, b2_ref,
              ps_ref, q0_ref, q1_ref, q2_ref, q3_ref, q4_ref):
    s = jnp.sum(s_ref[...], axis=0, keepdims=True)
    ss = jnp.sum(ss_ref[...], axis=0, keepdims=True)
    mean = s * (1.0 / N)
    var = ss * (1.0 / N) - mean * mean
    inv = lax.rsqrt(var + EPS)
    x1 = _lrelu((y1_ref[...] - mean) * inv)
    y = jnp.dot(x1, w_ref[...], preferred_element_type=jnp.float32)
    ps_ref[...] = y[:, 0:H] + b2_ref[...]
    q0_ref[...] = y[:, H:2 * H]
    q1_ref[...] = y[:, 2 * H:3 * H]
    q2_ref[...] = y[:, 3 * H:4 * H]
    q3_ref[...] = y[:, 4 * H:5 * H]
    q4_ref[...] = y[:, 5 * H:6 * H]


def _tc2(y1, s1, ss1, wbig2, b2):
    row_spec = pl.BlockSpec((T1, H), lambda i: (i, 0))
    return pl.pallas_call(
        _tc2_body,
        grid=(GRID,),
        in_specs=[
            pl.BlockSpec((T1, H), lambda i: (i, 0)),
            pl.BlockSpec((NW, H), lambda i: (0, 0)),
            pl.BlockSpec((NW, H), lambda i: (0, 0)),
            pl.BlockSpec((H, 6 * H), lambda i: (0, 0)),
            pl.BlockSpec((1, H), lambda i: (0, 0)),
        ],
        out_specs=[row_spec] * 6,
        out_shape=[jax.ShapeDtypeStruct((NPAD, H), jnp.float32)] * 6,
    )(y1, s1, ss1, wbig2, b2)


def _tc3_body(y2_ref, s2_ref, ss2_ref, r_ref, sr_ref, ssr_ref, w4_ref, o_ref):
    s2 = jnp.sum(s2_ref[...], axis=0, keepdims=True)
    ss2 = jnp.sum(ss2_ref[...], axis=0, keepdims=True)
    m2 = s2 * (1.0 / N)
    v2 = ss2 * (1.0 / N) - m2 * m2
    inv2 = lax.rsqrt(v2 + EPS)
    mr = sr_ref[...] * (1.0 / N)
    vr = ssr_ref[...] * (1.0 / N) - mr * mr
    invr = lax.rsqrt(vr + EPS)
    x = _lrelu((y2_ref[...] - m2) * inv2
               + (r_ref[...].astype(jnp.float32) - mr) * invr)
    o_ref[...] = jnp.sum(x * w4_ref[...], axis=1, keepdims=True).reshape(1, 1, T1)


def _tc3(y2, s2, ss2, r, sr, ssr, w4row):
    return pl.pallas_call(
        _tc3_body,
        grid=(GRID,),
        in_specs=[
            pl.BlockSpec((T1, H), lambda i: (i, 0)),
            pl.BlockSpec((NW, H), lambda i: (0, 0)),
            pl.BlockSpec((NW, H), lambda i: (0, 0)),
            pl.BlockSpec((T1, H), lambda i: (i, 0)),
            pl.BlockSpec((1, H), lambda i: (0, 0)),
            pl.BlockSpec((1, H), lambda i: (0, 0)),
            pl.BlockSpec((1, H), lambda i: (0, 0)),
        ],
        out_specs=pl.BlockSpec((1, 1, T1), lambda i: (i, 0, 0)),
        out_shape=jax.ShapeDtypeStruct((GRID, 1, T1), jnp.float32),
    )(y2, s2, ss2, r, sr, ssr, w4row)


def kernel(features, neighbors_index, W1, b1, W2, b2, W3, b3, W4):
    f = features[0]

    idx = neighbors_index[0].astype(jnp.int32).T
    idx = jnp.pad(idx, ((0, 0), (0, NPAD - N)))
    idxr = idx.reshape(K, NW, NCHUNK, CH).transpose(1, 2, 0, 3)

    wb1 = [W1[:, j * C:(j + 1) * C].T for j in range(K + 1)] + [W3.T]
    wbig1 = jnp.concatenate(wb1, axis=1)
    wb2 = [W2[:, j * H:(j + 1) * H].T for j in range(K + 1)]
    wbig2 = jnp.concatenate(wb2, axis=1)

    b1r = b1.reshape(1, H)
    b2r = b2.reshape(1, H)
    b3r = b3.reshape(1, H)
    w4r = W4.reshape(1, H)

    ps1, q0, q1, q2, q3, q4, r, sr, ssr = _tc1(f, wbig1, b1r, b3r)
    y1, s1, ss1 = _sc_gather_sum(ps1, q0, q1, q2, q3, q4, idxr)
    ps2, p0, p1, p2, p3, p4 = _tc2(y1, s1, ss1, wbig2, b2r)
    y2, s2, ss2 = _sc_gather_sum(ps2, p0, p1, p2, p3, p4, idxr)
    out = _tc3(y2, s2, ss2, r, sr, ssr, w4r)
    return out.reshape(1, N, 1)


# speedup vs baseline: 1.1271x; 1.0411x over previous
import functools

import jax
import jax.numpy as jnp
from jax import lax
from jax.experimental import pallas as pl
from jax.experimental.pallas import tpu as pltpu
from jax.experimental.pallas import tpu_sc as plsc

N = 50000
C = 256
K = 5
H = 128
EPS = 1e-5
SLOPE = 0.01

NC, NS = 2, 16
NW = NC * NS
CH = 64
ROWS_W = 1664
NCHUNK = ROWS_W // CH
NPAD = NW * ROWS_W

T1 = 1000
GRID = N // T1


def _lrelu(x):
    return jnp.where(x >= 0, x, SLOPE * x)


def _tc1_body(f_ref, w_ref, b1_ref, b3_ref,
              ps_ref, q0_ref, q1_ref, q2_ref, q3_ref, q4_ref, r_ref,
              sr_ref, ssr_ref):
    x = f_ref[...].astype(jnp.bfloat16)
    y = jnp.dot(x, w_ref[...], preferred_element_type=jnp.float32)
    ps_ref[...] = y[:, 0:H] + b1_ref[...]
    q0_ref[...] = y[:, H:2 * H]
    q1_ref[...] = y[:, 2 * H:3 * H]
    q2_ref[...] = y[:, 3 * H:4 * H]
    q3_ref[...] = y[:, 4 * H:5 * H]
    q4_ref[...] = y[:, 5 * H:6 * H]
    r = y[:, 6 * H:7 * H] + b3_ref[...]
    r_ref[...] = r.astype(jnp.bfloat16)

    i = pl.program_id(0)

    @pl.when(i == 0)
    def _():
        sr_ref[...] = jnp.zeros_like(sr_ref)
        ssr_ref[...] = jnp.zeros_like(ssr_ref)

    sr_ref[...] += jnp.sum(r, axis=0, keepdims=True)
    ssr_ref[...] += jnp.sum(r * r, axis=0, keepdims=True)


def _tc1(f, wbig1, b1, b3):
    row_spec = pl.BlockSpec((T1, H), lambda i: (i, 0))
    return pl.pallas_call(
        _tc1_body,
        grid=(GRID,),
        in_specs=[
            pl.BlockSpec((T1, C), lambda i: (i, 0)),
            pl.BlockSpec((C, 7 * H), lambda i: (0, 0)),
            pl.BlockSpec((1, H), lambda i: (0, 0)),
            pl.BlockSpec((1, H), lambda i: (0, 0)),
        ],
        out_specs=[row_spec] * 7 + [pl.BlockSpec((1, H), lambda i: (0, 0))] * 2,
        out_shape=[jax.ShapeDtypeStruct((NPAD, H), jnp.float32)] * 6
        + [jax.ShapeDtypeStruct((NPAD, H), jnp.bfloat16)]
        + [jax.ShapeDtypeStruct((1, H), jnp.float32)] * 2,
    )(f, wbig1, b1, b3)


def _sc_body(ps_hbm, t0, t1, t2, t3, t4, idx_hbm,
             y_hbm, sum_hbm, ssq_hbm,
             idx_v, bufs, sum_v, ssq_v, sem0, sem1):
    wid = lax.axis_index("s") * NC + lax.axis_index("c")
    base = wid * ROWS_W
    tables = (t0, t1, t2, t3, t4)
    sems = (sem0, sem1)
    NJ = H // 16

    pltpu.sync_copy(idx_hbm.at[wid], idx_v)

    def copies(c, s):
        row0 = base + c * CH
        ops = [pltpu.make_async_copy(tables[k].at[idx_v.at[c, k]],
                                     bufs.at[s, k], sems[s]) for k in range(K)]
        ops.append(pltpu.make_async_copy(ps_hbm.at[pl.ds(row0, CH)],
                                         bufs.at[s, K], sems[s]))
        return ops

    def issue(c, s):
        for op in copies(c, s):
            op.start()

    def compute(c, s, stats):
        row0 = base + c * CH
        for op in copies(c, s):
            op.wait()

        def row_body(r, st):
            valid = (row0 + r) < N
            sums, ssqs = st
            new_sums, new_ssqs = [], []
            for j in range(NJ):
                sl = pl.ds(16 * j, 16)
                s_ = bufs[s, K, r, sl]
                for k in range(K):
                    s_ = s_ + bufs[s, k, r, sl]
                bufs[s, K, r, sl] = s_
                sv = jnp.where(valid, s_, 0.0)
                new_sums.append(sums[j] + sv)
                new_ssqs.append(ssqs[j] + sv * sv)
            return (tuple(new_sums), tuple(new_ssqs))

        stats = plsc.parallel_loop(0, CH, 1, unroll=8, carry=stats)(row_body)
        pltpu.sync_copy(bufs.at[s, K], y_hbm.at[pl.ds(row0, CH)])
        return stats

    zeros = tuple(jnp.zeros((16,), jnp.float32) for _ in range(NJ))
    stats0 = (zeros, zeros)

    issue(0, 0)
    issue(1, 1)

    def pair_body(i, stats):
        c = 2 * i
        stats = compute(c, 0, stats)

        @pl.when(c + 2 < NCHUNK)
        def _():
            issue(c + 2, 0)

        stats = compute(c + 1, 1, stats)

        @pl.when(c + 3 < NCHUNK)
        def _():
            issue(c + 3, 1)

        return stats

    stats = lax.fori_loop(0, NCHUNK // 2, pair_body, stats0)
    sums, ssqs = stats
    for j in range(NJ):
        sl = pl.ds(16 * j, 16)
        sum_v[sl] = sums[j]
        ssq_v[sl] = ssqs[j]
    pltpu.sync_copy(sum_v, sum_hbm.at[wid])
    pltpu.sync_copy(ssq_v, ssq_hbm.at[wid])


_sc_gather_sum = functools.partial(
    pl.kernel,
    out_type=[
        jax.ShapeDtypeStruct((NPAD, H), jnp.float32),
        jax.ShapeDtypeStruct((NW, H), jnp.float32),
        jax.ShapeDtypeStruct((NW, H), jnp.float32),
    ],
    mesh=plsc.VectorSubcoreMesh(core_axis_name="c", subcore_axis_name="s"),
    scratch_types=[
        pltpu.VMEM((NCHUNK, K, CH), jnp.int32),
        pltpu.VMEM((2, K + 1, CH, H), jnp.float32),
        pltpu.VMEM((H,), jnp.float32),
        pltpu.VMEM((H,), jnp.float32),
        pltpu.SemaphoreType.DMA,
        pltpu.SemaphoreType.DMA,
    ],
)(_sc_body)


def _tc2_body(y1_ref, s_ref, ss_ref, w_ref, b2_ref,
              ps_ref, q0_ref, q1_ref, q2_ref, q3_ref, q4_ref):
    s = jnp.sum(s_ref[...], axis=0, keepdims=True)
    ss = jnp.sum(ss_ref[...], axis=0, keepdims=True)
    mean = s * (1.0 / N)
    var = ss * (1.0 / N) - mean * mean
    inv = lax.rsqrt(var + EPS)
    x1 = _lrelu((y1_ref[...] - mean) * inv).astype(jnp.bfloat16)
    y = jnp.dot(x1, w_ref[...], preferred_element_type=jnp.float32)
    ps_ref[...] = y[:, 0:H] + b2_ref[...]
    q0_ref[...] = y[:, H:2 * H]
    q1_ref[...] = y[:, 2 * H:3 * H]
    q2_ref[...] = y[:, 3 * H:4 * H]
    q3_ref[...] = y[:, 4 * H:5 * H]
    q4_ref[...] = y[:, 5 * H:6 * H]


def _tc2(y1, s1, ss1, wbig2, b2):
    row_spec = pl.BlockSpec((T1, H), lambda i: (i, 0))
    return pl.pallas_call(
        _tc2_body,
        grid=(GRID,),
        in_specs=[
            pl.BlockSpec((T1, H), lambda i: (i, 0)),
            pl.BlockSpec((NW, H), lambda i: (0, 0)),
            pl.BlockSpec((NW, H), lambda i: (0, 0)),
            pl.BlockSpec((H, 6 * H), lambda i: (0, 0)),
            pl.BlockSpec((1, H), lambda i: (0, 0)),
        ],
        out_specs=[row_spec] * 6,
        out_shape=[jax.ShapeDtypeStruct((NPAD, H), jnp.float32)] * 6,
    )(y1, s1, ss1, wbig2, b2)


def _tc3_body(y2_ref, s2_ref, ss2_ref, r_ref, sr_ref, ssr_ref, w4_ref, o_ref):
    s2 = jnp.sum(s2_ref[...], axis=0, keepdims=True)
    ss2 = jnp.sum(ss2_ref[...], axis=0, keepdims=True)
    m2 = s2 * (1.0 / N)
    v2 = ss2 * (1.0 / N) - m2 * m2
    inv2 = lax.rsqrt(v2 + EPS)
    mr = sr_ref[...] * (1.0 / N)
    vr = ssr_ref[...] * (1.0 / N) - mr * mr
    invr = lax.rsqrt(vr + EPS)
    x = _lrelu((y2_ref[...] - m2) * inv2
               + (r_ref[...].astype(jnp.float32) - mr) * invr)
    o_ref[...] = jnp.sum(x * w4_ref[...], axis=1, keepdims=True).reshape(1, 1, T1)


def _tc3(y2, s2, ss2, r, sr, ssr, w4row):
    return pl.pallas_call(
        _tc3_body,
        grid=(GRID,),
        in_specs=[
            pl.BlockSpec((T1, H), lambda i: (i, 0)),
            pl.BlockSpec((NW, H), lambda i: (0, 0)),
            pl.BlockSpec((NW, H), lambda i: (0, 0)),
            pl.BlockSpec((T1, H), lambda i: (i, 0)),
            pl.BlockSpec((1, H), lambda i: (0, 0)),
            pl.BlockSpec((1, H), lambda i: (0, 0)),
            pl.BlockSpec((1, H), lambda i: (0, 0)),
        ],
        out_specs=pl.BlockSpec((1, 1, T1), lambda i: (i, 0, 0)),
        out_shape=jax.ShapeDtypeStruct((GRID, 1, T1), jnp.float32),
    )(y2, s2, ss2, r, sr, ssr, w4row)


def kernel(features, neighbors_index, W1, b1, W2, b2, W3, b3, W4):
    f = features[0]

    idx = neighbors_index[0].astype(jnp.int32).T
    idx = jnp.pad(idx, ((0, 0), (0, NPAD - N)))
    idxr = idx.reshape(K, NW, NCHUNK, CH).transpose(1, 2, 0, 3)

    wb1 = [W1[:, j * C:(j + 1) * C].T for j in range(K + 1)] + [W3.T]
    wbig1 = jnp.concatenate(wb1, axis=1).astype(jnp.bfloat16)
    wb2 = [W2[:, j * H:(j + 1) * H].T for j in range(K + 1)]
    wbig2 = jnp.concatenate(wb2, axis=1).astype(jnp.bfloat16)

    b1r = b1.reshape(1, H)
    b2r = b2.reshape(1, H)
    b3r = b3.reshape(1, H)
    w4r = W4.reshape(1, H)

    ps1, q0, q1, q2, q3, q4, r, sr, ssr = _tc1(f, wbig1, b1r, b3r)
    y1, s1, ss1 = _sc_gather_sum(ps1, q0, q1, q2, q3, q4, idxr)
    ps2, p0, p1, p2, p3, p4 = _tc2(y1, s1, ss1, wbig2, b2r)
    y2, s2, ss2 = _sc_gather_sum(ps2, p0, p1, p2, p3, p4, idxr)
    out = _tc3(y2, s2, ss2, r, sr, ssr, w4r)
    return out.reshape(1, N, 1)


# speedup vs baseline: 1.2003x; 1.0649x over previous
import functools

import jax
import jax.numpy as jnp
from jax import lax
from jax.experimental import pallas as pl
from jax.experimental.pallas import tpu as pltpu
from jax.experimental.pallas import tpu_sc as plsc

N = 50000
C = 256
K = 5
H = 128
EPS = 1e-5
SLOPE = 0.01

NC, NS = 2, 16
NW = NC * NS
CH = 64
ROWS_W = 1664
NCHUNK = ROWS_W // CH
NPAD = NW * ROWS_W

T1 = 2000
GRID = N // T1


def _lrelu(x):
    return jnp.where(x >= 0, x, SLOPE * x)


def _tc1_body(f_ref, w_ref, b1_ref, b3_ref,
              ps_ref, q0_ref, q1_ref, q2_ref, q3_ref, q4_ref, r_ref,
              sr_ref, ssr_ref):
    x = f_ref[...].astype(jnp.bfloat16)
    y = jnp.dot(x, w_ref[...], preferred_element_type=jnp.float32)
    ps_ref[...] = y[:, 0:H] + b1_ref[...]
    q0_ref[...] = y[:, H:2 * H]
    q1_ref[...] = y[:, 2 * H:3 * H]
    q2_ref[...] = y[:, 3 * H:4 * H]
    q3_ref[...] = y[:, 4 * H:5 * H]
    q4_ref[...] = y[:, 5 * H:6 * H]
    r = y[:, 6 * H:7 * H] + b3_ref[...]
    r_ref[...] = r.astype(jnp.bfloat16)

    i = pl.program_id(0)

    @pl.when(i == 0)
    def _():
        sr_ref[...] = jnp.zeros_like(sr_ref)
        ssr_ref[...] = jnp.zeros_like(ssr_ref)

    sr_ref[...] += jnp.sum(r, axis=0, keepdims=True)
    ssr_ref[...] += jnp.sum(r * r, axis=0, keepdims=True)


def _tc1(f, wbig1, b1, b3):
    row_spec = pl.BlockSpec((T1, H), lambda i: (i, 0))
    return pl.pallas_call(
        _tc1_body,
        grid=(GRID,),
        in_specs=[
            pl.BlockSpec((T1, C), lambda i: (i, 0)),
            pl.BlockSpec((C, 7 * H), lambda i: (0, 0)),
            pl.BlockSpec((1, H), lambda i: (0, 0)),
            pl.BlockSpec((1, H), lambda i: (0, 0)),
        ],
        out_specs=[row_spec] * 7 + [pl.BlockSpec((1, H), lambda i: (0, 0))] * 2,
        out_shape=[jax.ShapeDtypeStruct((NPAD, H), jnp.float32)] * 6
        + [jax.ShapeDtypeStruct((NPAD, H), jnp.bfloat16)]
        + [jax.ShapeDtypeStruct((1, H), jnp.float32)] * 2,
    )(f, wbig1, b1, b3)


def _sc_body(ps_hbm, t0, t1, t2, t3, t4, idx_hbm,
             y_hbm, sum_hbm, ssq_hbm,
             idx_v, bufs, sum_v, ssq_v, sem0, sem1):
    wid = lax.axis_index("s") * NC + lax.axis_index("c")
    base = wid * ROWS_W
    tables = (t0, t1, t2, t3, t4)
    sems = (sem0, sem1)
    NJ = H // 16

    pltpu.sync_copy(idx_hbm.at[wid], idx_v)

    def copies(c, s):
        row0 = base + c * CH
        ops = [pltpu.make_async_copy(tables[k].at[idx_v.at[c, k]],
                                     bufs.at[s, k], sems[s]) for k in range(K)]
        ops.append(pltpu.make_async_copy(ps_hbm.at[pl.ds(row0, CH)],
                                         bufs.at[s, K], sems[s]))
        return ops

    def issue(c, s):
        for op in copies(c, s):
            op.start()

    def compute(c, s, stats):
        row0 = base + c * CH
        for op in copies(c, s):
            op.wait()

        def row_body(r, st):
            valid = (row0 + r) < N
            sums, ssqs = st
            new_sums, new_ssqs = [], []
            for j in range(NJ):
                sl = pl.ds(16 * j, 16)
                s_ = bufs[s, K, r, sl]
                for k in range(K):
                    s_ = s_ + bufs[s, k, r, sl]
                bufs[s, K, r, sl] = s_
                sv = jnp.where(valid, s_, 0.0)
                new_sums.append(sums[j] + sv)
                new_ssqs.append(ssqs[j] + sv * sv)
            return (tuple(new_sums), tuple(new_ssqs))

        stats = plsc.parallel_loop(0, CH, 1, unroll=8, carry=stats)(row_body)
        pltpu.sync_copy(bufs.at[s, K], y_hbm.at[pl.ds(row0, CH)])
        return stats

    zeros = tuple(jnp.zeros((16,), jnp.float32) for _ in range(NJ))
    stats0 = (zeros, zeros)

    issue(0, 0)
    issue(1, 1)

    def pair_body(i, stats):
        c = 2 * i
        stats = compute(c, 0, stats)

        @pl.when(c + 2 < NCHUNK)
        def _():
            issue(c + 2, 0)

        stats = compute(c + 1, 1, stats)

        @pl.when(c + 3 < NCHUNK)
        def _():
            issue(c + 3, 1)

        return stats

    stats = lax.fori_loop(0, NCHUNK // 2, pair_body, stats0)
    sums, ssqs = stats
    for j in range(NJ):
        sl = pl.ds(16 * j, 16)
        sum_v[sl] = sums[j]
        ssq_v[sl] = ssqs[j]
    pltpu.sync_copy(sum_v, sum_hbm.at[wid])
    pltpu.sync_copy(ssq_v, ssq_hbm.at[wid])


_sc_gather_sum = functools.partial(
    pl.kernel,
    out_type=[
        jax.ShapeDtypeStruct((NPAD, H), jnp.float32),
        jax.ShapeDtypeStruct((NW, H), jnp.float32),
        jax.ShapeDtypeStruct((NW, H), jnp.float32),
    ],
    mesh=plsc.VectorSubcoreMesh(core_axis_name="c", subcore_axis_name="s"),
    scratch_types=[
        pltpu.VMEM((NCHUNK, K, CH), jnp.int32),
        pltpu.VMEM((2, K + 1, CH, H), jnp.float32),
        pltpu.VMEM((H,), jnp.float32),
        pltpu.VMEM((H,), jnp.float32),
        pltpu.SemaphoreType.DMA,
        pltpu.SemaphoreType.DMA,
    ],
)(_sc_body)


def _tc2_body(y1_ref, s_ref, ss_ref, w_ref, b2_ref,
              ps_ref, q0_ref, q1_ref, q2_ref, q3_ref, q4_ref):
    s = jnp.sum(s_ref[...], axis=0, keepdims=True)
    ss = jnp.sum(ss_ref[...], axis=0, keepdims=True)
    mean = s * (1.0 / N)
    var = ss * (1.0 / N) - mean * mean
    inv = lax.rsqrt(var + EPS)
    x1 = _lrelu((y1_ref[...] - mean) * inv).astype(jnp.bfloat16)
    y = jnp.dot(x1, w_ref[...], preferred_element_type=jnp.float32)
    ps_ref[...] = y[:, 0:H] + b2_ref[...]
    q0_ref[...] = y[:, H:2 * H]
    q1_ref[...] = y[:, 2 * H:3 * H]
    q2_ref[...] = y[:, 3 * H:4 * H]
    q3_ref[...] = y[:, 4 * H:5 * H]
    q4_ref[...] = y[:, 5 * H:6 * H]


def _tc2(y1, s1, ss1, wbig2, b2):
    row_spec = pl.BlockSpec((T1, H), lambda i: (i, 0))
    return pl.pallas_call(
        _tc2_body,
        grid=(GRID,),
        in_specs=[
            pl.BlockSpec((T1, H), lambda i: (i, 0)),
            pl.BlockSpec((NW, H), lambda i: (0, 0)),
            pl.BlockSpec((NW, H), lambda i: (0, 0)),
            pl.BlockSpec((H, 6 * H), lambda i: (0, 0)),
            pl.BlockSpec((1, H), lambda i: (0, 0)),
        ],
        out_specs=[row_spec] * 6,
        out_shape=[jax.ShapeDtypeStruct((NPAD, H), jnp.float32)] * 6,
    )(y1, s1, ss1, wbig2, b2)


def _tc3_body(y2_ref, s2_ref, ss2_ref, r_ref, sr_ref, ssr_ref, w4_ref, o_ref):
    s2 = jnp.sum(s2_ref[...], axis=0, keepdims=True)
    ss2 = jnp.sum(ss2_ref[...], axis=0, keepdims=True)
    m2 = s2 * (1.0 / N)
    v2 = ss2 * (1.0 / N) - m2 * m2
    inv2 = lax.rsqrt(v2 + EPS)
    mr = sr_ref[...] * (1.0 / N)
    vr = ssr_ref[...] * (1.0 / N) - mr * mr
    invr = lax.rsqrt(vr + EPS)
    x = _lrelu((y2_ref[...] - m2) * inv2
               + (r_ref[...].astype(jnp.float32) - mr) * invr)
    o_ref[...] = jnp.sum(x * w4_ref[...], axis=1, keepdims=True).reshape(1, 1, T1)


def _tc3(y2, s2, ss2, r, sr, ssr, w4row):
    return pl.pallas_call(
        _tc3_body,
        grid=(GRID,),
        in_specs=[
            pl.BlockSpec((T1, H), lambda i: (i, 0)),
            pl.BlockSpec((NW, H), lambda i: (0, 0)),
            pl.BlockSpec((NW, H), lambda i: (0, 0)),
            pl.BlockSpec((T1, H), lambda i: (i, 0)),
            pl.BlockSpec((1, H), lambda i: (0, 0)),
            pl.BlockSpec((1, H), lambda i: (0, 0)),
            pl.BlockSpec((1, H), lambda i: (0, 0)),
        ],
        out_specs=pl.BlockSpec((1, 1, T1), lambda i: (i, 0, 0)),
        out_shape=jax.ShapeDtypeStruct((GRID, 1, T1), jnp.float32),
    )(y2, s2, ss2, r, sr, ssr, w4row)


def kernel(features, neighbors_index, W1, b1, W2, b2, W3, b3, W4):
    f = features[0]

    idx = neighbors_index[0].astype(jnp.int32).T
    idx = jnp.pad(idx, ((0, 0), (0, NPAD - N)))
    idxr = idx.reshape(K, NW, NCHUNK, CH).transpose(1, 2, 0, 3)

    wb1 = [W1[:, j * C:(j + 1) * C].T for j in range(K + 1)] + [W3.T]
    wbig1 = jnp.concatenate(wb1, axis=1).astype(jnp.bfloat16)
    wb2 = [W2[:, j * H:(j + 1) * H].T for j in range(K + 1)]
    wbig2 = jnp.concatenate(wb2, axis=1).astype(jnp.bfloat16)

    b1r = b1.reshape(1, H)
    b2r = b2.reshape(1, H)
    b3r = b3.reshape(1, H)
    w4r = W4.reshape(1, H)

    ps1, q0, q1, q2, q3, q4, r, sr, ssr = _tc1(f, wbig1, b1r, b3r)
    y1, s1, ss1 = _sc_gather_sum(ps1, q0, q1, q2, q3, q4, idxr)
    ps2, p0, p1, p2, p3, p4 = _tc2(y1, s1, ss1, wbig2, b2r)
    y2, s2, ss2 = _sc_gather_sum(ps2, p0, p1, p2, p3, p4, idxr)
    out = _tc3(y2, s2, ss2, r, sr, ssr, w4r)
    return out.reshape(1, N, 1)


# speedup vs baseline: 1.2191x; 1.0157x over previous
import functools

import jax
import jax.numpy as jnp
from jax import lax
from jax.experimental import pallas as pl
from jax.experimental.pallas import tpu as pltpu
from jax.experimental.pallas import tpu_sc as plsc

N = 50000
C = 256
K = 5
H = 128
EPS = 1e-5
SLOPE = 0.01

NC, NS = 2, 16
NW = NC * NS
CH = 64
ROWS_W = 1664
NCHUNK = ROWS_W // CH
NPAD = NW * ROWS_W

T1 = 5000
GRID = N // T1


def _lrelu(x):
    return jnp.where(x >= 0, x, SLOPE * x)


def _tc1_body(f_ref, w_ref, b1_ref, b3_ref,
              ps_ref, q0_ref, q1_ref, q2_ref, q3_ref, q4_ref, r_ref,
              sr_ref, ssr_ref):
    x = f_ref[...].astype(jnp.bfloat16)
    y = jnp.dot(x, w_ref[...], preferred_element_type=jnp.float32)
    ps_ref[...] = y[:, 0:H] + b1_ref[...]
    q0_ref[...] = y[:, H:2 * H]
    q1_ref[...] = y[:, 2 * H:3 * H]
    q2_ref[...] = y[:, 3 * H:4 * H]
    q3_ref[...] = y[:, 4 * H:5 * H]
    q4_ref[...] = y[:, 5 * H:6 * H]
    r = y[:, 6 * H:7 * H] + b3_ref[...]
    r_ref[...] = r.astype(jnp.bfloat16)

    i = pl.program_id(0)

    @pl.when(i == 0)
    def _():
        sr_ref[...] = jnp.zeros_like(sr_ref)
        ssr_ref[...] = jnp.zeros_like(ssr_ref)

    sr_ref[...] += jnp.sum(r, axis=0, keepdims=True)
    ssr_ref[...] += jnp.sum(r * r, axis=0, keepdims=True)


def _tc1(f, wbig1, b1, b3):
    row_spec = pl.BlockSpec((T1, H), lambda i: (i, 0))
    return pl.pallas_call(
        _tc1_body,
        grid=(GRID,),
        in_specs=[
            pl.BlockSpec((T1, C), lambda i: (i, 0)),
            pl.BlockSpec((C, 7 * H), lambda i: (0, 0)),
            pl.BlockSpec((1, H), lambda i: (0, 0)),
            pl.BlockSpec((1, H), lambda i: (0, 0)),
        ],
        out_specs=[row_spec] * 7 + [pl.BlockSpec((1, H), lambda i: (0, 0))] * 2,
        out_shape=[jax.ShapeDtypeStruct((NPAD, H), jnp.float32)] * 6
        + [jax.ShapeDtypeStruct((NPAD, H), jnp.bfloat16)]
        + [jax.ShapeDtypeStruct((1, H), jnp.float32)] * 2,
    )(f, wbig1, b1, b3)


def _sc_body(ps_hbm, t0, t1, t2, t3, t4, idx_hbm,
             y_hbm, sum_hbm, ssq_hbm,
             idx_v, bufs, sum_v, ssq_v, sem0, sem1):
    wid = lax.axis_index("s") * NC + lax.axis_index("c")
    base = wid * ROWS_W
    tables = (t0, t1, t2, t3, t4)
    sems = (sem0, sem1)
    NJ = H // 16

    pltpu.sync_copy(idx_hbm.at[wid], idx_v)

    def copies(c, s):
        row0 = base + c * CH
        ops = [pltpu.make_async_copy(tables[k].at[idx_v.at[c, k]],
                                     bufs.at[s, k], sems[s]) for k in range(K)]
        ops.append(pltpu.make_async_copy(ps_hbm.at[pl.ds(row0, CH)],
                                         bufs.at[s, K], sems[s]))
        return ops

    def issue(c, s):
        for op in copies(c, s):
            op.start()

    def compute(c, s, stats):
        row0 = base + c * CH
        for op in copies(c, s):
            op.wait()

        def row_body(r, st):
            valid = (row0 + r) < N
            sums, ssqs = st
            new_sums, new_ssqs = [], []
            for j in range(NJ):
                sl = pl.ds(16 * j, 16)
                s_ = bufs[s, K, r, sl]
                for k in range(K):
                    s_ = s_ + bufs[s, k, r, sl]
                bufs[s, K, r, sl] = s_
                sv = jnp.where(valid, s_, 0.0)
                new_sums.append(sums[j] + sv)
                new_ssqs.append(ssqs[j] + sv * sv)
            return (tuple(new_sums), tuple(new_ssqs))

        stats = plsc.parallel_loop(0, CH, 1, unroll=8, carry=stats)(row_body)
        pltpu.sync_copy(bufs.at[s, K], y_hbm.at[pl.ds(row0, CH)])
        return stats

    zeros = tuple(jnp.zeros((16,), jnp.float32) for _ in range(NJ))
    stats0 = (zeros, zeros)

    issue(0, 0)
    issue(1, 1)

    def pair_body(i, stats):
        c = 2 * i
        stats = compute(c, 0, stats)

        @pl.when(c + 2 < NCHUNK)
        def _():
            issue(c + 2, 0)

        stats = compute(c + 1, 1, stats)

        @pl.when(c + 3 < NCHUNK)
        def _():
            issue(c + 3, 1)

        return stats

    stats = lax.fori_loop(0, NCHUNK // 2, pair_body, stats0)
    sums, ssqs = stats
    for j in range(NJ):
        sl = pl.ds(16 * j, 16)
        sum_v[sl] = sums[j]
        ssq_v[sl] = ssqs[j]
    pltpu.sync_copy(sum_v, sum_hbm.at[wid])
    pltpu.sync_copy(ssq_v, ssq_hbm.at[wid])


_sc_gather_sum = functools.partial(
    pl.kernel,
    out_type=[
        jax.ShapeDtypeStruct((NPAD, H), jnp.float32),
        jax.ShapeDtypeStruct((NW, H), jnp.float32),
        jax.ShapeDtypeStruct((NW, H), jnp.float32),
    ],
    mesh=plsc.VectorSubcoreMesh(core_axis_name="c", subcore_axis_name="s"),
    scratch_types=[
        pltpu.VMEM((NCHUNK, K, CH), jnp.int32),
        pltpu.VMEM((2, K + 1, CH, H), jnp.float32),
        pltpu.VMEM((H,), jnp.float32),
        pltpu.VMEM((H,), jnp.float32),
        pltpu.SemaphoreType.DMA,
        pltpu.SemaphoreType.DMA,
    ],
)(_sc_body)


def _tc2_body(y1_ref, s_ref, ss_ref, w_ref, b2_ref,
              ps_ref, q0_ref, q1_ref, q2_ref, q3_ref, q4_ref):
    s = jnp.sum(s_ref[...], axis=0, keepdims=True)
    ss = jnp.sum(ss_ref[...], axis=0, keepdims=True)
    mean = s * (1.0 / N)
    var = ss * (1.0 / N) - mean * mean
    inv = lax.rsqrt(var + EPS)
    x1 = _lrelu((y1_ref[...] - mean) * inv).astype(jnp.bfloat16)
    y = jnp.dot(x1, w_ref[...], preferred_element_type=jnp.float32)
    ps_ref[...] = y[:, 0:H] + b2_ref[...]
    q0_ref[...] = y[:, H:2 * H]
    q1_ref[...] = y[:, 2 * H:3 * H]
    q2_ref[...] = y[:, 3 * H:4 * H]
    q3_ref[...] = y[:, 4 * H:5 * H]
    q4_ref[...] = y[:, 5 * H:6 * H]


def _tc2(y1, s1, ss1, wbig2, b2):
    row_spec = pl.BlockSpec((T1, H), lambda i: (i, 0))
    return pl.pallas_call(
        _tc2_body,
        grid=(GRID,),
        in_specs=[
            pl.BlockSpec((T1, H), lambda i: (i, 0)),
            pl.BlockSpec((NW, H), lambda i: (0, 0)),
            pl.BlockSpec((NW, H), lambda i: (0, 0)),
            pl.BlockSpec((H, 6 * H), lambda i: (0, 0)),
            pl.BlockSpec((1, H), lambda i: (0, 0)),
        ],
        out_specs=[row_spec] * 6,
        out_shape=[jax.ShapeDtypeStruct((NPAD, H), jnp.float32)] * 6,
    )(y1, s1, ss1, wbig2, b2)


def _tc3_body(y2_ref, s2_ref, ss2_ref, r_ref, sr_ref, ssr_ref, w4_ref, o_ref):
    s2 = jnp.sum(s2_ref[...], axis=0, keepdims=True)
    ss2 = jnp.sum(ss2_ref[...], axis=0, keepdims=True)
    m2 = s2 * (1.0 / N)
    v2 = ss2 * (1.0 / N) - m2 * m2
    inv2 = lax.rsqrt(v2 + EPS)
    mr = sr_ref[...] * (1.0 / N)
    vr = ssr_ref[...] * (1.0 / N) - mr * mr
    invr = lax.rsqrt(vr + EPS)
    x = _lrelu((y2_ref[...] - m2) * inv2
               + (r_ref[...].astype(jnp.float32) - mr) * invr)
    o_ref[...] = jnp.sum(x * w4_ref[...], axis=1, keepdims=True).reshape(1, 1, T1)


def _tc3(y2, s2, ss2, r, sr, ssr, w4row):
    return pl.pallas_call(
        _tc3_body,
        grid=(GRID,),
        in_specs=[
            pl.BlockSpec((T1, H), lambda i: (i, 0)),
            pl.BlockSpec((NW, H), lambda i: (0, 0)),
            pl.BlockSpec((NW, H), lambda i: (0, 0)),
            pl.BlockSpec((T1, H), lambda i: (i, 0)),
            pl.BlockSpec((1, H), lambda i: (0, 0)),
            pl.BlockSpec((1, H), lambda i: (0, 0)),
            pl.BlockSpec((1, H), lambda i: (0, 0)),
        ],
        out_specs=pl.BlockSpec((1, 1, T1), lambda i: (i, 0, 0)),
        out_shape=jax.ShapeDtypeStruct((GRID, 1, T1), jnp.float32),
    )(y2, s2, ss2, r, sr, ssr, w4row)


def kernel(features, neighbors_index, W1, b1, W2, b2, W3, b3, W4):
    f = features[0]

    idx = neighbors_index[0].astype(jnp.int32).T
    idx = jnp.pad(idx, ((0, 0), (0, NPAD - N)))
    idxr = idx.reshape(K, NW, NCHUNK, CH).transpose(1, 2, 0, 3)

    wb1 = [W1[:, j * C:(j + 1) * C].T for j in range(K + 1)] + [W3.T]
    wbig1 = jnp.concatenate(wb1, axis=1).astype(jnp.bfloat16)
    wb2 = [W2[:, j * H:(j + 1) * H].T for j in range(K + 1)]
    wbig2 = jnp.concatenate(wb2, axis=1).astype(jnp.bfloat16)

    b1r = b1.reshape(1, H)
    b2r = b2.reshape(1, H)
    b3r = b3.reshape(1, H)
    w4r = W4.reshape(1, H)

    ps1, q0, q1, q2, q3, q4, r, sr, ssr = _tc1(f, wbig1, b1r, b3r)
    y1, s1, ss1 = _sc_gather_sum(ps1, q0, q1, q2, q3, q4, idxr)
    ps2, p0, p1, p2, p3, p4 = _tc2(y1, s1, ss1, wbig2, b2r)
    y2, s2, ss2 = _sc_gather_sum(ps2, p0, p1, p2, p3, p4, idxr)
    out = _tc3(y2, s2, ss2, r, sr, ssr, w4r)
    return out.reshape(1, N, 1)
